# Initial kernel scaffold; baseline (speedup 1.0000x reference)
import jax
import jax.numpy as jnp
from jax.experimental import pallas as pl


def kernel(x, act_table, mode_table):
    raise NotImplementedError("write your pallas kernel here")


# speedup vs baseline: 2.0461x; 2.0461x over previous
import functools

import jax
import jax.numpy as jnp
from jax import lax
from jax.experimental import pallas as pl
from jax.experimental.pallas import tpu as pltpu
from jax.experimental.pallas import tpu_sc as plsc

NC = 2
NS = 16
NW = NC * NS
LANES = 16

B, L = 4096, 200
N = B * L
DA, DM = 64, 16
DO = DA + 1 + DM + 1

TOK_PER_W = N // NW
T = 512
CHUNKS = TOK_PER_W // T
NIDX = T // 128


def _body(x_hbm, act_hbm, mode_hbm, out_hbm,
          x_v, idxa_v, idxm_v, acts_v, modes_v, out_v, sem):
  wid = lax.axis_index("s") * NC + lax.axis_index("c")
  tok0 = wid * TOK_PER_W

  iota = lax.iota(jnp.int32, LANES)
  iota4 = iota * 4
  iota82 = iota * DO

  def chunk(i, carry):
    base = tok0 + i * T

    pltpu.sync_copy(x_hbm.at[pl.ds(base * 4, T * 4)], x_v)

    for j in range(T // LANES):
      xb = iota4 + (j * 4 * LANES)
      a_f = plsc.load_gather(x_v, [xb])
      d_f = plsc.load_gather(x_v, [xb + 1])
      m_f = plsc.load_gather(x_v, [xb + 2])
      s_f = plsc.load_gather(x_v, [xb + 3])
      row = jnp.full((LANES,), j // 8, dtype=jnp.int32)
      col = iota + ((j % 8) * LANES)
      plsc.store_scatter(idxa_v, [row, col], a_f.astype(jnp.int32))
      plsc.store_scatter(idxm_v, [row, col], m_f.astype(jnp.int32))
      pos = iota82 + (j * LANES * DO)
      plsc.store_scatter(out_v, [pos + DA], d_f)
      plsc.store_scatter(out_v, [pos + (DO - 1)], s_f)

    copies = []
    for j in range(NIDX):
      copies.append(pltpu.async_copy(
          act_hbm.at[idxa_v.at[j]], acts_v.at[pl.ds(j * 128, 128)], sem))
      copies.append(pltpu.async_copy(
          mode_hbm.at[idxm_v.at[j]], modes_v.at[pl.ds(j * 128, 128)], sem))
    for c in copies:
      c.wait()

    def asm(t8, carry2):
      t0 = t8 * 8
      for k in range(8):
        t = t0 + k
        rowt = jnp.full((LANES,), t, dtype=jnp.int32)
        obase = t * DO
        for c4 in range(DA // LANES):
          v = plsc.load_gather(acts_v, [rowt, iota + (c4 * LANES)])
          plsc.store_scatter(out_v, [iota + (obase + c4 * LANES)], v)
        m = plsc.load_gather(modes_v, [rowt, iota])
        plsc.store_scatter(out_v, [iota + (obase + DA + 1)], m)
      return carry2

    lax.fori_loop(0, T // 8, asm, 0, unroll=False)

    pltpu.sync_copy(out_v, out_hbm.at[pl.ds(base * DO, T * DO)])
    return carry

  lax.fori_loop(0, CHUNKS, chunk, 0, unroll=False)


@jax.jit
def kernel(x, act_table, mode_table):
  mesh = plsc.VectorSubcoreMesh(
      core_axis_name="c", subcore_axis_name="s",
      num_cores=NC, num_subcores=NS)
  k = pl.kernel(
      _body,
      out_type=jax.ShapeDtypeStruct((N * DO,), jnp.float32),
      mesh=mesh,
      scratch_types=[
          pltpu.VMEM((T * 4,), jnp.float32),
          pltpu.VMEM((NIDX, 128), jnp.int32),
          pltpu.VMEM((NIDX, 128), jnp.int32),
          pltpu.VMEM((T, DA), jnp.float32),
          pltpu.VMEM((T, DM), jnp.float32),
          pltpu.VMEM((T * DO,), jnp.float32),
          pltpu.SemaphoreType.DMA,
      ],
      compiler_params=pltpu.CompilerParams(
          needs_layout_passes=False, use_tc_tiling_on_sc=False),
  )
  out = k(x.reshape(N * 4), act_table, mode_table)
  return out.reshape(B, L, DO)


# speedup vs baseline: 2.5623x; 1.2522x over previous
import functools

import jax
import jax.numpy as jnp
from jax import lax
from jax.experimental import pallas as pl
from jax.experimental.pallas import tpu as pltpu
from jax.experimental.pallas import tpu_sc as plsc

NC = 2
NS = 16
NW = NC * NS
LANES = 16

B, L = 4096, 200
N = B * L
DA, DM = 64, 16
DO = DA + 1 + DM + 1
DT = DO - DA

TOK_PER_W = N // NW
T = 512
CHUNKS = TOK_PER_W // T
NIDX = T // 128


def _body(x_hbm, act_hbm, mode_hbm, out_hbm,
          x_v, idxa_v, idxm_v, acts_v, modes_v, dm_v, sem):
  wid = lax.axis_index("s") * NC + lax.axis_index("c")
  tok0 = wid * TOK_PER_W

  iota = lax.iota(jnp.int32, LANES)
  iota4 = iota * 4
  c_dur = jnp.full((LANES,), 0, dtype=jnp.int32)
  c_dist = jnp.full((LANES,), DT - 1, dtype=jnp.int32)

  def chunk(i, carry):
    base = tok0 + i * T

    pltpu.sync_copy(x_hbm.at[pl.ds(base * 4, T * 4)], x_v)

    for j in range(T // LANES):
      xb = iota4 + (j * 4 * LANES)
      a_f = plsc.load_gather(x_v, [xb])
      d_f = plsc.load_gather(x_v, [xb + 1])
      m_f = plsc.load_gather(x_v, [xb + 2])
      s_f = plsc.load_gather(x_v, [xb + 3])
      row = jnp.full((LANES,), j // 8, dtype=jnp.int32)
      col = iota + ((j % 8) * LANES)
      plsc.store_scatter(idxa_v, [row, col], a_f.astype(jnp.int32))
      plsc.store_scatter(idxm_v, [row, col], m_f.astype(jnp.int32))
      tok = iota + (j * LANES)
      plsc.store_scatter(dm_v, [tok, c_dur], d_f)
      plsc.store_scatter(dm_v, [tok, c_dist], s_f)

    copies = []
    for j in range(NIDX):
      rows = pl.ds(j * 128, 128)
      copies.append(pltpu.async_copy(
          act_hbm.at[idxa_v.at[j]], acts_v.at[rows], sem))
      copies.append(pltpu.async_copy(
          mode_hbm.at[idxm_v.at[j]], modes_v.at[rows], sem))
    for c in copies:
      c.wait()

    def asm(t8, carry2):
      t0 = t8 * 8
      for k in range(8):
        rowt = jnp.full((LANES,), t0 + k, dtype=jnp.int32)
        m = plsc.load_gather(modes_v, [rowt, iota])
        plsc.store_scatter(dm_v, [rowt, iota + 1], m)
      return carry2

    lax.fori_loop(0, T // 8, asm, 0, unroll=False)

    rows_hbm = pl.ds(base, T)
    pltpu.sync_copy(acts_v, out_hbm.at[rows_hbm, pl.ds(0, DA)])
    pltpu.sync_copy(dm_v, out_hbm.at[rows_hbm, pl.ds(DA, DT)])
    return carry

  lax.fori_loop(0, CHUNKS, chunk, 0, unroll=False)


@jax.jit
def kernel(x, act_table, mode_table):
  mesh = plsc.VectorSubcoreMesh(
      core_axis_name="c", subcore_axis_name="s",
      num_cores=NC, num_subcores=NS)
  k = pl.kernel(
      _body,
      out_type=jax.ShapeDtypeStruct((N, DO), jnp.float32),
      mesh=mesh,
      scratch_types=[
          pltpu.VMEM((T * 4,), jnp.float32),
          pltpu.VMEM((NIDX, 128), jnp.int32),
          pltpu.VMEM((NIDX, 128), jnp.int32),
          pltpu.VMEM((T, DA), jnp.float32),
          pltpu.VMEM((T, DM), jnp.float32),
          pltpu.VMEM((T, DT), jnp.float32),
          pltpu.SemaphoreType.DMA,
      ],
      compiler_params=pltpu.CompilerParams(
          needs_layout_passes=False, use_tc_tiling_on_sc=False),
  )
  out = k(x.reshape(N * 4), act_table, mode_table)
  return out.reshape(B, L, DO)
